# Initial kernel scaffold; baseline (speedup 1.0000x reference)
import jax
import jax.numpy as jnp
from jax.experimental import pallas as pl


def kernel(x, T0, scale0, shift0, T1, scale1, shift1, T2, scale2, shift2, T3, scale3, shift3, dense_w8, dense_b8):
    raise NotImplementedError("write your pallas kernel here")


# speedup vs baseline: 5.5394x; 5.5394x over previous
import jax
import jax.numpy as jnp
from jax.experimental import pallas as pl
from jax.experimental.pallas import tpu as pltpu

_B = 8
_H = 71
_R = _H * _B
_VMEM = 100 * 1024 * 1024


def _fused_body(x_ref, t0_ref, s0_ref, h0_ref, t1_ref, s1_ref, h1_ref,
                t2_ref, s2_ref, h2_ref, t3_ref, s3_ref, h3_ref,
                w2_ref, m_ref, bd_ref, o_ref):
    f32 = jnp.float32

    def conv(xp, t_ref, s_ref, h_ref):
        x5 = jnp.concatenate([xp[8 * kh: 8 * kh + _R, :] for kh in range(5)],
                             axis=1)
        y = jnp.dot(x5, t_ref[...], preferred_element_type=f32)
        y = y * s_ref[...] + h_ref[...]
        return jnp.maximum(y, 0.0)

    def repad(y, rpad):
        return jnp.pad(y, ((16, 16), (0, rpad)))

    y = conv(x_ref[0], t0_ref, s0_ref, h0_ref)
    y = jnp.maximum(y[:, :160], y[:, 160:])
    y = conv(repad(y, 96), t1_ref, s1_ref, h1_ref)
    y = jnp.maximum(y[:, :160], y[:, 160:])
    y = conv(repad(y, 96), t2_ref, s2_ref, h2_ref)
    e = conv(repad(y, 64), t3_ref, s3_ref, h3_ref)

    g = jnp.dot(e, w2_ref[...], preferred_element_type=f32)
    g = g * m_ref[...]
    t2 = g.reshape(_H, _B, _R).sum(axis=0)
    yv = t2.reshape(_B, _H, _B).sum(axis=1)
    o_ref[...] = yv + bd_ref[...]


def _pool_perm(t, wo, cout):
    kh, k, _ = t.shape
    t = t.reshape(kh, k, wo // 2, 2, cout)
    t = jnp.transpose(t, (0, 1, 3, 2, 4))
    return t.reshape(kh, k, wo * cout)


def _pack_t(t, cin, keep_lo, keep_hi, kpad, perm_wo=None, cout=None):
    t = t[:, keep_lo * cin: keep_hi * cin, :]
    if perm_wo is not None:
        t = _pool_perm(t, perm_wo, cout)
    t = jnp.pad(t, ((0, 0), (0, kpad - t.shape[1]), (0, 0)))
    return t.reshape(5 * kpad, t.shape[2])


def kernel(x, T0, scale0, shift0, T1, scale1, shift1, T2, scale2, shift2,
           T3, scale3, shift3, dense_w8, dense_b8):
    n = x.shape[0]
    nb = n // _B
    xh = jnp.pad(x.reshape(n, _H, 40), ((0, 0), (2, 2), (0, 0)))
    xh = xh.reshape(nb, _B, _H + 4, 40).transpose(0, 2, 1, 3)
    xh = xh.reshape(nb, (_H + 4) * _B, 40)

    t0 = _pack_t(T0, 2, 2, 22, 40, perm_wo=20, cout=16)
    t1 = _pack_t(T1, 16, 2, 12, 256, perm_wo=10, cout=32)
    t2 = _pack_t(T2, 32, 2, 7, 256)
    t3 = _pack_t(T3, 64, 1, 6, 384)

    wd = dense_w8.reshape(8, _H, 640)
    w2 = wd.transpose(2, 1, 0).reshape(640, _R)
    ri = jax.lax.broadcasted_iota(jnp.int32, (_R, _R), 0) // _B
    ci = jax.lax.broadcasted_iota(jnp.int32, (_R, _R), 1) // _B
    mask = (ri == ci).astype(jnp.float32)

    const2 = lambda i: (0, 0)
    out = pl.pallas_call(
        _fused_body,
        out_shape=jax.ShapeDtypeStruct((n, 8), jnp.float32),
        grid=(nb,),
        in_specs=[
            pl.BlockSpec((1, (_H + 4) * _B, 40), lambda i: (i, 0, 0)),
            pl.BlockSpec(t0.shape, const2),
            pl.BlockSpec(scale0.shape, const2),
            pl.BlockSpec(shift0.shape, const2),
            pl.BlockSpec(t1.shape, const2),
            pl.BlockSpec(scale1.shape, const2),
            pl.BlockSpec(shift1.shape, const2),
            pl.BlockSpec(t2.shape, const2),
            pl.BlockSpec(scale2.shape, const2),
            pl.BlockSpec(shift2.shape, const2),
            pl.BlockSpec(t3.shape, const2),
            pl.BlockSpec(scale3.shape, const2),
            pl.BlockSpec(shift3.shape, const2),
            pl.BlockSpec(w2.shape, const2),
            pl.BlockSpec(mask.shape, const2),
            pl.BlockSpec(dense_b8.shape, const2),
        ],
        out_specs=pl.BlockSpec((_B, 8), lambda i: (i, 0)),
        compiler_params=pltpu.CompilerParams(
            dimension_semantics=("parallel",),
            vmem_limit_bytes=_VMEM),
    )(xh, t0, scale0, shift0, t1, scale1, shift1,
      t2, scale2, shift2, t3, scale3, shift3, w2, mask, dense_b8)
    return out[:, :7]


# speedup vs baseline: 5.5405x; 1.0002x over previous
import jax
import jax.numpy as jnp
from jax.experimental import pallas as pl
from jax.experimental.pallas import tpu as pltpu

_B = 16
_H = 71
_R = _H * _B
_GN = _H * 8
_VMEM = 100 * 1024 * 1024
_BF = jnp.bfloat16


def _fused_body(x_ref, t0_ref, s0_ref, h0_ref, t1_ref, s1_ref, h1_ref,
                t2_ref, s2_ref, h2_ref, t3_ref, s3_ref, h3_ref,
                w2_ref, m_ref, bd_ref, o_ref):
    f32 = jnp.float32

    def conv(xp, t_ref, s_ref, h_ref):
        x5 = jnp.concatenate(
            [xp[_B * kh: _B * kh + _R, :] for kh in range(5)], axis=1)
        y = jnp.dot(x5, t_ref[...], preferred_element_type=f32)
        y = y * s_ref[...] + h_ref[...]
        return jnp.maximum(y, 0.0)

    def repad(y, rpad):
        return jnp.pad(y.astype(_BF), ((2 * _B, 2 * _B), (0, rpad)))

    y = conv(x_ref[0], t0_ref, s0_ref, h0_ref)
    y = jnp.maximum(y[:, :160], y[:, 160:])
    y = conv(repad(y, 96), t1_ref, s1_ref, h1_ref)
    y = jnp.maximum(y[:, :160], y[:, 160:])
    y = conv(repad(y, 96), t2_ref, s2_ref, h2_ref)
    e = conv(repad(y, 64), t3_ref, s3_ref, h3_ref)

    g = jnp.dot(e.astype(_BF), w2_ref[...],
                preferred_element_type=f32)
    g = g * m_ref[...]
    t2 = g.reshape(_H, _B, _GN).sum(axis=0)
    yv = t2.reshape(_B, _H, 8).sum(axis=1)
    o_ref[...] = yv + bd_ref[...]


def _pool_perm(t, wo, cout):
    kh, k, _ = t.shape
    t = t.reshape(kh, k, wo // 2, 2, cout)
    t = jnp.transpose(t, (0, 1, 3, 2, 4))
    return t.reshape(kh, k, wo * cout)


def _pack_t(t, cin, keep_lo, keep_hi, kpad, perm_wo=None, cout=None):
    t = t[:, keep_lo * cin: keep_hi * cin, :]
    if perm_wo is not None:
        t = _pool_perm(t, perm_wo, cout)
    t = jnp.pad(t, ((0, 0), (0, kpad - t.shape[1]), (0, 0)))
    return t.reshape(5 * kpad, t.shape[2]).astype(_BF)


def kernel(x, T0, scale0, shift0, T1, scale1, shift1, T2, scale2, shift2,
           T3, scale3, shift3, dense_w8, dense_b8):
    n = x.shape[0]
    nb = n // _B
    xh = jnp.pad(x.reshape(n, _H, 40), ((0, 0), (2, 2), (0, 0)))
    xh = xh.reshape(nb, _B, _H + 4, 40).transpose(0, 2, 1, 3)
    xh = xh.reshape(nb, (_H + 4) * _B, 40).astype(_BF)

    t0 = _pack_t(T0, 2, 2, 22, 40, perm_wo=20, cout=16)
    t1 = _pack_t(T1, 16, 2, 12, 256, perm_wo=10, cout=32)
    t2 = _pack_t(T2, 32, 2, 7, 256)
    t3 = _pack_t(T3, 64, 1, 6, 384)

    wd = dense_w8.reshape(8, _H, 640)
    w2 = wd.transpose(2, 1, 0).reshape(640, _GN).astype(_BF)
    ri = jax.lax.broadcasted_iota(jnp.int32, (_R, _GN), 0) // _B
    ci = jax.lax.broadcasted_iota(jnp.int32, (_R, _GN), 1) // 8
    mask = (ri == ci).astype(jnp.float32)

    const2 = lambda i: (0, 0)
    out = pl.pallas_call(
        _fused_body,
        out_shape=jax.ShapeDtypeStruct((n, 8), jnp.float32),
        grid=(nb,),
        in_specs=[
            pl.BlockSpec((1, (_H + 4) * _B, 40), lambda i: (i, 0, 0)),
            pl.BlockSpec(t0.shape, const2),
            pl.BlockSpec(scale0.shape, const2),
            pl.BlockSpec(shift0.shape, const2),
            pl.BlockSpec(t1.shape, const2),
            pl.BlockSpec(scale1.shape, const2),
            pl.BlockSpec(shift1.shape, const2),
            pl.BlockSpec(t2.shape, const2),
            pl.BlockSpec(scale2.shape, const2),
            pl.BlockSpec(shift2.shape, const2),
            pl.BlockSpec(t3.shape, const2),
            pl.BlockSpec(scale3.shape, const2),
            pl.BlockSpec(shift3.shape, const2),
            pl.BlockSpec(w2.shape, const2),
            pl.BlockSpec(mask.shape, const2),
            pl.BlockSpec(dense_b8.shape, const2),
        ],
        out_specs=pl.BlockSpec((_B, 8), lambda i: (i, 0)),
        compiler_params=pltpu.CompilerParams(
            dimension_semantics=("parallel",),
            vmem_limit_bytes=_VMEM),
    )(xh, t0, scale0, shift0, t1, scale1, shift1,
      t2, scale2, shift2, t3, scale3, shift3, w2, mask, dense_b8)
    return out[:, :7]


# speedup vs baseline: 5.9674x; 1.0771x over previous
import jax
import jax.numpy as jnp
from jax.experimental import pallas as pl
from jax.experimental.pallas import tpu as pltpu

_B = 32
_H = 71
_R = _H * _B
_GN = _H * 8
_VMEM = 100 * 1024 * 1024
_BF = jnp.bfloat16


def _fused_body(x_ref, t0_ref, h0_ref, t1_ref, h1_ref,
                t2_ref, h2_ref, t3_ref, h3_ref,
                w2_ref, m_ref, bd_ref, o_ref):
    f32 = jnp.float32

    def conv(xp, t_ref, h_ref):
        x5 = jnp.concatenate(
            [xp[_B * kh: _B * kh + _R, :] for kh in range(5)], axis=1)
        y = jnp.dot(x5, t_ref[...], preferred_element_type=f32)
        y = y + h_ref[...]
        return jnp.maximum(y, 0.0)

    def repad(y, rpad):
        return jnp.pad(y.astype(_BF), ((2 * _B, 2 * _B), (0, rpad)))

    y = conv(x_ref[0], t0_ref, h0_ref)
    y = jnp.maximum(y[:, :160], y[:, 160:])
    y = conv(repad(y, 96), t1_ref, h1_ref)
    y = jnp.maximum(y[:, :160], y[:, 160:])
    y = conv(repad(y, 96), t2_ref, h2_ref)
    e = conv(repad(y, 64), t3_ref, h3_ref)

    g = jnp.dot(e.astype(_BF), w2_ref[...],
                preferred_element_type=f32)
    g = g * m_ref[...]
    t2 = g.reshape(_H, _B, _GN).sum(axis=0)
    yv = t2.reshape(_B, _H, 8).sum(axis=1)
    o_ref[...] = yv + bd_ref[...]


def _pool_perm(t, wo, cout):
    kh, k, _ = t.shape
    t = t.reshape(kh, k, wo // 2, 2, cout)
    t = jnp.transpose(t, (0, 1, 3, 2, 4))
    return t.reshape(kh, k, wo * cout)


def _pack_t(t, scale, cin, keep_lo, keep_hi, kpad, perm_wo=None, cout=None):
    t = t[:, keep_lo * cin: keep_hi * cin, :] * scale[None]
    if perm_wo is not None:
        t = _pool_perm(t, perm_wo, cout)
    t = jnp.pad(t, ((0, 0), (0, kpad - t.shape[1]), (0, 0)))
    return t.reshape(5 * kpad, t.shape[2]).astype(_BF)


def kernel(x, T0, scale0, shift0, T1, scale1, shift1, T2, scale2, shift2,
           T3, scale3, shift3, dense_w8, dense_b8):
    n = x.shape[0]
    nb = n // _B
    xh = jnp.pad(x.reshape(n, _H, 40), ((0, 0), (2, 2), (0, 0)))
    xh = xh.reshape(nb, _B, _H + 4, 40).transpose(0, 2, 1, 3)
    xh = xh.reshape(nb, (_H + 4) * _B, 40).astype(_BF)

    t0 = _pack_t(T0, scale0, 2, 2, 22, 40, perm_wo=20, cout=16)
    t1 = _pack_t(T1, scale1, 16, 2, 12, 256, perm_wo=10, cout=32)
    t2 = _pack_t(T2, scale2, 32, 2, 7, 256)
    t3 = _pack_t(T3, scale3, 64, 1, 6, 384)

    wd = dense_w8.reshape(8, _H, 640)
    w2 = wd.transpose(2, 1, 0).reshape(640, _GN).astype(_BF)
    ri = jax.lax.broadcasted_iota(jnp.int32, (_R, _GN), 0) // _B
    ci = jax.lax.broadcasted_iota(jnp.int32, (_R, _GN), 1) // 8
    mask = (ri == ci).astype(jnp.float32)

    const2 = lambda i: (0, 0)
    out = pl.pallas_call(
        _fused_body,
        out_shape=jax.ShapeDtypeStruct((n, 8), jnp.float32),
        grid=(nb,),
        in_specs=[
            pl.BlockSpec((1, (_H + 4) * _B, 40), lambda i: (i, 0, 0)),
            pl.BlockSpec(t0.shape, const2),
            pl.BlockSpec(shift0.shape, const2),
            pl.BlockSpec(t1.shape, const2),
            pl.BlockSpec(shift1.shape, const2),
            pl.BlockSpec(t2.shape, const2),
            pl.BlockSpec(shift2.shape, const2),
            pl.BlockSpec(t3.shape, const2),
            pl.BlockSpec(shift3.shape, const2),
            pl.BlockSpec(w2.shape, const2),
            pl.BlockSpec(mask.shape, const2),
            pl.BlockSpec(dense_b8.shape, const2),
        ],
        out_specs=pl.BlockSpec((_B, 8), lambda i: (i, 0)),
        compiler_params=pltpu.CompilerParams(
            dimension_semantics=("parallel",),
            vmem_limit_bytes=_VMEM),
    )(xh, t0, shift0, t1, shift1,
      t2, shift2, t3, shift3, w2, mask, dense_b8)
    return out[:, :7]


# speedup vs baseline: 6.8733x; 1.1518x over previous
import jax
import jax.numpy as jnp
from jax.experimental import pallas as pl
from jax.experimental.pallas import tpu as pltpu

_B = 32
_H = 71
_R = _H * _B
_GN = _H * 8
_VMEM = 100 * 1024 * 1024
_BF = jnp.bfloat16


def _fused_body(x_ref, t0_ref, h0_ref, t1_ref, h1_ref,
                t2_ref, h2_ref, t3_ref, h3_ref,
                w2_ref, m_ref, bd_ref, o_ref):
    f32 = jnp.float32

    def conv(xp, t_ref, h_ref):
        x5 = jnp.concatenate(
            [xp[_B * kh: _B * kh + _R, :] for kh in range(5)], axis=1)
        y = jnp.dot(x5, t_ref[...], preferred_element_type=f32)
        y = y + h_ref[...]
        return jnp.maximum(y, 0.0)

    def repad(y, rpad):
        return jnp.pad(y.astype(_BF), ((2 * _B, 2 * _B), (0, rpad)))

    y = conv(x_ref[0], t0_ref, h0_ref)
    y = jnp.maximum(y[:, :160], y[:, 160:])
    y = conv(repad(y, 0), t1_ref, h1_ref)
    y = jnp.maximum(y[:, :160], y[:, 160:])
    y = conv(repad(y, 0), t2_ref, h2_ref)
    e = conv(repad(y, 0), t3_ref, h3_ref)

    g = jnp.dot(e.astype(_BF), w2_ref[...],
                preferred_element_type=f32)
    g = g * m_ref[...]
    t2 = g.reshape(_H, _B, _GN).sum(axis=0)
    yv = t2.reshape(_B, _H, 8).sum(axis=1)
    o_ref[...] = yv + bd_ref[...]


def _pool_perm(t, wo, cout):
    kh, k, _ = t.shape
    t = t.reshape(kh, k, wo // 2, 2, cout)
    t = jnp.transpose(t, (0, 1, 3, 2, 4))
    return t.reshape(kh, k, wo * cout)


def _pack_t(t, scale, cin, keep_lo, keep_hi, kpad, perm_wo=None, cout=None):
    t = t[:, keep_lo * cin: keep_hi * cin, :] * scale[None]
    if perm_wo is not None:
        t = _pool_perm(t, perm_wo, cout)
    t = jnp.pad(t, ((0, 0), (0, kpad - t.shape[1]), (0, 0)))
    return t.reshape(5 * kpad, t.shape[2]).astype(_BF)


def kernel(x, T0, scale0, shift0, T1, scale1, shift1, T2, scale2, shift2,
           T3, scale3, shift3, dense_w8, dense_b8):
    n = x.shape[0]
    nb = n // _B
    xh = jnp.pad(x.reshape(n, _H, 40), ((0, 0), (2, 2), (0, 0)))
    xh = xh.reshape(nb, _B, _H + 4, 40).transpose(0, 2, 1, 3)
    xh = xh.reshape(nb, (_H + 4) * _B, 40).astype(_BF)

    t0 = _pack_t(T0, scale0, 2, 2, 22, 40, perm_wo=20, cout=16)
    t1 = _pack_t(T1, scale1, 16, 2, 12, 160, perm_wo=10, cout=32)
    t2 = _pack_t(T2, scale2, 32, 2, 7, 160)
    t3 = _pack_t(T3, scale3, 64, 1, 6, 320)

    wd = dense_w8.reshape(8, _H, 640)
    w2 = wd.transpose(2, 1, 0).reshape(640, _GN).astype(_BF)
    ri = jax.lax.broadcasted_iota(jnp.int32, (_R, _GN), 0) // _B
    ci = jax.lax.broadcasted_iota(jnp.int32, (_R, _GN), 1) // 8
    mask = (ri == ci).astype(jnp.float32)

    const2 = lambda i: (0, 0)
    out = pl.pallas_call(
        _fused_body,
        out_shape=jax.ShapeDtypeStruct((n, 8), jnp.float32),
        grid=(nb,),
        in_specs=[
            pl.BlockSpec((1, (_H + 4) * _B, 40), lambda i: (i, 0, 0)),
            pl.BlockSpec(t0.shape, const2),
            pl.BlockSpec(shift0.shape, const2),
            pl.BlockSpec(t1.shape, const2),
            pl.BlockSpec(shift1.shape, const2),
            pl.BlockSpec(t2.shape, const2),
            pl.BlockSpec(shift2.shape, const2),
            pl.BlockSpec(t3.shape, const2),
            pl.BlockSpec(shift3.shape, const2),
            pl.BlockSpec(w2.shape, const2),
            pl.BlockSpec(mask.shape, const2),
            pl.BlockSpec(dense_b8.shape, const2),
        ],
        out_specs=pl.BlockSpec((_B, 8), lambda i: (i, 0)),
        compiler_params=pltpu.CompilerParams(
            dimension_semantics=("parallel",),
            vmem_limit_bytes=_VMEM),
    )(xh, t0, shift0, t1, shift1,
      t2, shift2, t3, shift3, w2, mask, dense_b8)
    return out[:, :7]


# speedup vs baseline: 7.0246x; 1.0220x over previous
import jax
import jax.numpy as jnp
from jax.experimental import pallas as pl
from jax.experimental.pallas import tpu as pltpu

_B = 32
_H = 71
_R = _H * _B
_GN = _H * 8
_VMEM = 100 * 1024 * 1024
_BF = jnp.bfloat16


def _fused_body(x_ref, t0_ref, h0_ref, t1_ref, h1_ref,
                t2_ref, h2_ref, t3_ref, h3_ref,
                w2_ref, m_ref, bd_ref, o_ref):
    f32 = jnp.float32

    def conv(xp, t_ref, h_ref, pool=False):
        x5 = jnp.concatenate(
            [xp[_B * kh: _B * kh + _R, :] for kh in range(5)], axis=1)
        y = jnp.dot(x5, t_ref[...], preferred_element_type=f32)
        if pool:
            h = y.shape[1] // 2
            y = jnp.maximum(y[:, :h], y[:, h:])
        return jnp.maximum(y + h_ref[...], 0.0)

    def repad(y, rpad):
        return jnp.pad(y.astype(_BF), ((2 * _B, 2 * _B), (0, rpad)))

    y = conv(x_ref[0], t0_ref, h0_ref, pool=True)
    y = conv(repad(y, 0), t1_ref, h1_ref, pool=True)
    y = conv(repad(y, 0), t2_ref, h2_ref)
    e = conv(repad(y, 0), t3_ref, h3_ref)

    g = jnp.dot(e.astype(_BF), w2_ref[...],
                preferred_element_type=f32)
    g = g * m_ref[...]
    t2 = g.reshape(_H, _B, _GN).sum(axis=0)
    yv = t2.reshape(_B, _H, 8).sum(axis=1)
    o_ref[...] = yv + bd_ref[...]


def _pool_perm(t, wo, cout):
    kh, k, _ = t.shape
    t = t.reshape(kh, k, wo // 2, 2, cout)
    t = jnp.transpose(t, (0, 1, 3, 2, 4))
    return t.reshape(kh, k, wo * cout)


def _pack_t(t, scale, cin, keep_lo, keep_hi, kpad, perm_wo=None, cout=None):
    t = t[:, keep_lo * cin: keep_hi * cin, :] * scale[None]
    if perm_wo is not None:
        t = _pool_perm(t, perm_wo, cout)
    t = jnp.pad(t, ((0, 0), (0, kpad - t.shape[1]), (0, 0)))
    return t.reshape(5 * kpad, t.shape[2]).astype(_BF)


def kernel(x, T0, scale0, shift0, T1, scale1, shift1, T2, scale2, shift2,
           T3, scale3, shift3, dense_w8, dense_b8):
    n = x.shape[0]
    nb = n // _B
    xh = jnp.pad(x.reshape(n, _H, 40), ((0, 0), (2, 2), (0, 0)))
    xh = xh.reshape(nb, _B, _H + 4, 40).transpose(0, 2, 1, 3)
    xh = xh.reshape(nb, (_H + 4) * _B, 40).astype(_BF)

    t0 = _pack_t(T0, scale0, 2, 2, 22, 40, perm_wo=20, cout=16)
    t1 = _pack_t(T1, scale1, 16, 2, 12, 160, perm_wo=10, cout=32)
    t2 = _pack_t(T2, scale2, 32, 2, 7, 160)
    t3 = _pack_t(T3, scale3, 64, 1, 6, 320)

    wd = dense_w8.reshape(8, _H, 640)
    w2 = wd.transpose(2, 1, 0).reshape(640, _GN).astype(_BF)
    ri = jax.lax.broadcasted_iota(jnp.int32, (_R, _GN), 0) // _B
    ci = jax.lax.broadcasted_iota(jnp.int32, (_R, _GN), 1) // 8
    mask = (ri == ci).astype(jnp.float32)

    const2 = lambda i: (0, 0)
    out = pl.pallas_call(
        _fused_body,
        out_shape=jax.ShapeDtypeStruct((n, 8), jnp.float32),
        grid=(nb,),
        in_specs=[
            pl.BlockSpec((1, (_H + 4) * _B, 40), lambda i: (i, 0, 0)),
            pl.BlockSpec(t0.shape, const2),
            pl.BlockSpec((1, 160), const2),
            pl.BlockSpec(t1.shape, const2),
            pl.BlockSpec((1, 160), const2),
            pl.BlockSpec(t2.shape, const2),
            pl.BlockSpec(shift2.shape, const2),
            pl.BlockSpec(t3.shape, const2),
            pl.BlockSpec(shift3.shape, const2),
            pl.BlockSpec(w2.shape, const2),
            pl.BlockSpec(mask.shape, const2),
            pl.BlockSpec(dense_b8.shape, const2),
        ],
        out_specs=pl.BlockSpec((_B, 8), lambda i: (i, 0)),
        compiler_params=pltpu.CompilerParams(
            dimension_semantics=("parallel",),
            vmem_limit_bytes=_VMEM),
    )(xh, t0, shift0[:, :160], t1, shift1[:, :160],
      t2, shift2, t3, shift3, w2, mask, dense_b8)
    return out[:, :7]
